# Initial kernel scaffold; baseline (speedup 1.0000x reference)
import jax
import jax.numpy as jnp
from jax.experimental import pallas as pl


def kernel(doc_sents_he, doc_sents_hc, pred_emo, pos_emb_weight, emo_emb_weight):
    raise NotImplementedError("write your pallas kernel here")


# speedup vs baseline: 1.8164x; 1.8164x over previous
import numpy as np
import jax
import jax.numpy as jnp
from jax import lax
from jax.experimental import pallas as pl
from jax.experimental.pallas import tpu as pltpu
from jax.experimental.pallas import tpu_sc as plsc

B = 16
S = 64
K = 3
F = 384
EDIM = 32
PDIM = 32
TAGS = 7
OUTW = 2 * F + EDIM + PDIM

_base = np.arange(1, S + 1)
_emo = np.repeat(_base, S)
_cau = np.tile(_base, S)
_rel = _cau - _emo
_msk = np.abs(_rel) <= K
I_P = (_emo[_msk] - 1).astype(np.int32)
J_P = (_cau[_msk] - 1).astype(np.int32)
NPAIR = int(I_P.shape[0])
HALF = NPAIR // 2
EMO_CAU = np.stack([_emo[_msk], _cau[_msk]], axis=1).astype(np.int32)

_u = np.arange(2 * K + 1)
_counts = (S - np.abs(_u - K)).astype(np.float64)
COEFF = (np.exp(-((_u[:, None] - _u[None, :]) ** 2).astype(np.float64))
         * _counts[None, :]).astype(np.float32)

NW = 32
NCH = 2
CH = 112
PADN = NCH * CH
NVALID = (CH, HALF - CH)


def _pad_half(a):
    return np.concatenate([a, np.full(PADN - HALF, a[-1], a.dtype)])


GIDX = np.zeros((NW, NCH, CH), np.int32)
CIDX = np.zeros((NW, NCH, CH), np.int32)
ILOC = np.zeros((NW, PADN), np.int32)
for _w in range(NW):
    _b, _h = _w // 2, _w % 2
    _sl = slice(_h * HALF, (_h + 1) * HALF)
    GIDX[_w] = _pad_half(_b * S + I_P[_sl]).reshape(NCH, CH)
    CIDX[_w] = _pad_half(_b * S + J_P[_sl]).reshape(NCH, CH)
    ILOC[_w] = _pad_half(I_P[_sl])


def _rel_groups(h):
    out, pb = [], 0
    for i in range(h * (S // 2), (h + 1) * (S // 2)):
        lo, hi = max(0, i - K), min(S - 1, i + K)
        n = hi - lo + 1
        out.append((pb, lo - i + K, n))
        pb += n
    return tuple(out)


REL_GROUPS = (_rel_groups(0), _rel_groups(1))


def _sc_body(he, hc, pe, pos, etab, gidx, cidx, iloc, out,
             gidxv, cidxv, ilocv, emidxv, rowbuf, embuf, predv, predids,
             posv, relbuf, sem):
    cid = lax.axis_index("c")
    sid = lax.axis_index("s")
    wid = sid * 2 + cid
    b = wid // 2
    h = wid % 2
    row0 = b * NPAIR + h * HALF

    pltpu.sync_copy(gidx.at[wid], gidxv)
    pltpu.sync_copy(cidx.at[wid], cidxv)
    pltpu.sync_copy(iloc.at[wid], ilocv)
    pltpu.sync_copy(pe.at[b], predv)
    pltpu.sync_copy(pos, posv)

    for c in range(NCH):
        n = NVALID[c]
        pltpu.async_copy(he.at[gidxv.at[c]], rowbuf, sem).wait()
        pltpu.sync_copy(rowbuf.at[pl.ds(0, n)],
                        out.at[pl.ds(row0 + c * CH, n), pl.ds(0, F)])
        pltpu.async_copy(hc.at[cidxv.at[c]], rowbuf, sem).wait()
        pltpu.sync_copy(rowbuf.at[pl.ds(0, n)],
                        out.at[pl.ds(row0 + c * CH, n), pl.ds(F, F)])

    lanes7 = jnp.arange(16, dtype=jnp.int32) * TAGS
    for g in range(S // 16):
        addr0 = lanes7 + (g * 16 * TAGS)
        bv = plsc.load_gather(predv, [addr0])
        bi = jnp.zeros(16, jnp.int32)
        for t in range(1, TAGS):
            v = plsc.load_gather(predv, [addr0 + t])
            gt = v > bv
            bi = jnp.where(gt, jnp.int32(t), bi)
            bv = jnp.where(gt, v, bv)
        predids[pl.ds(g * 16, 16)] = bi

    for k in range(PADN // 16):
        iv = ilocv[pl.ds(k * 16, 16)]
        emidxv[pl.ds(k * 16, 16)] = plsc.load_gather(predids, [iv])
    for c in range(NCH):
        n = NVALID[c]
        pltpu.async_copy(etab.at[emidxv.at[pl.ds(c * CH, CH)]], embuf,
                         sem).wait()
        pltpu.sync_copy(embuf.at[pl.ds(0, n)],
                        out.at[pl.ds(row0 + c * CH, n), pl.ds(2 * F, EDIM)])

    for u in range(TAGS):
        for c2 in range(PDIM // 16):
            acc = COEFF[u, 0] * posv[pl.ds(c2 * 16, 16)]
            for v in range(1, TAGS):
                acc = acc + COEFF[u, v] * posv[pl.ds(v * PDIM + c2 * 16, 16)]
            relbuf[u, pl.ds(c2 * 16, 16)] = acc

    for h_ in range(2):
        @pl.when(h == h_)
        def _():
            descs = []
            for (pb, a, n) in REL_GROUPS[h_]:
                descs.append(pltpu.async_copy(
                    relbuf.at[pl.ds(a, n)],
                    out.at[pl.ds(row0 + pb, n), pl.ds(2 * F + EDIM, PDIM)],
                    sem))
            for d in descs:
                d.wait()


def kernel(doc_sents_he, doc_sents_hc, pred_emo, pos_emb_weight,
           emo_emb_weight):
    he2 = doc_sents_he.reshape(B * S, F)
    hc2 = doc_sents_hc.reshape(B * S, F)
    pe2 = pred_emo.reshape(B, S * TAGS)
    pos2 = pos_emb_weight.reshape(TAGS * PDIM)

    mesh = plsc.VectorSubcoreMesh(core_axis_name="c", subcore_axis_name="s")
    kfn = pl.kernel(
        _sc_body,
        out_type=jax.ShapeDtypeStruct((B * NPAIR, OUTW), jnp.float32),
        mesh=mesh,
        scratch_types=[
            pltpu.VMEM((NCH, CH), jnp.int32),
            pltpu.VMEM((NCH, CH), jnp.int32),
            pltpu.VMEM((PADN,), jnp.int32),
            pltpu.VMEM((PADN,), jnp.int32),
            pltpu.VMEM((CH, F), jnp.float32),
            pltpu.VMEM((CH, EDIM), jnp.float32),
            pltpu.VMEM((S * TAGS,), jnp.float32),
            pltpu.VMEM((S,), jnp.int32),
            pltpu.VMEM((TAGS * PDIM,), jnp.float32),
            pltpu.VMEM((TAGS, PDIM), jnp.float32),
            pltpu.SemaphoreType.DMA,
        ],
        compiler_params=pltpu.CompilerParams(use_tc_tiling_on_sc=False,
                                             needs_layout_passes=False),
    )
    outflat = kfn(he2, hc2, pe2, pos2, emo_emb_weight,
                  jnp.asarray(GIDX), jnp.asarray(CIDX), jnp.asarray(ILOC))
    couples = outflat.reshape(B, NPAIR, OUTW)
    return (couples, jnp.asarray(EMO_CAU))


# speedup vs baseline: 2.8772x; 1.5840x over previous
import numpy as np
import jax
import jax.numpy as jnp
from jax import lax
from jax.experimental import pallas as pl
from jax.experimental.pallas import tpu as pltpu
from jax.experimental.pallas import tpu_sc as plsc

B = 16
S = 64
K = 3
F = 384
EDIM = 32
PDIM = 32
TAGS = 7
OUTW = 2 * F + EDIM + PDIM
TAIL0 = 2 * F

_base = np.arange(1, S + 1)
_emo = np.repeat(_base, S)
_cau = np.tile(_base, S)
_rel = _cau - _emo
_msk = np.abs(_rel) <= K
I_P = (_emo[_msk] - 1).astype(np.int32)
J_P = (_cau[_msk] - 1).astype(np.int32)
R_P = (_rel[_msk] + K).astype(np.int32)
NPAIR = int(I_P.shape[0])
EMO_CAU = np.stack([_emo[_msk], _cau[_msk]], axis=1).astype(np.int32)

_u = np.arange(2 * K + 1)
_counts = (S - np.abs(_u - K)).astype(np.float64)
COEFF = (np.exp(-((_u[:, None] - _u[None, :]) ** 2).astype(np.float64))
         * _counts[None, :])

SEL_E = np.zeros((NPAIR, S), np.float32)
SEL_E[np.arange(NPAIR), I_P] = 1.0
_selr = np.zeros((NPAIR, TAGS), np.float64)
_selr[np.arange(NPAIR), R_P] = 1.0
SELR_COEFF = (_selr @ COEFF).astype(np.float32)

NW = 32
CH = 112
C1 = 104
WROWS = CH + C1
SC_ROWS = 2 * WROWS
PADW = 2 * CH


def _pad_to(a, n):
    return np.concatenate([a, np.full(n - a.shape[0], a[-1], a.dtype)])


GIDX = np.zeros((NW, 2, CH), np.int32)
CIDX = np.zeros((NW, 2, CH), np.int32)
for _w in range(NW):
    _b, _h = _w // 2, _w % 2
    _sl = slice(_h * WROWS, (_h + 1) * WROWS)
    GIDX[_w] = _pad_to(_b * S + I_P[_sl], PADW).reshape(2, CH)
    CIDX[_w] = _pad_to(_b * S + J_P[_sl], PADW).reshape(2, CH)


def _sc_body(he, hc, gidx, cidx, out, gidxv, cidxv, bufa, bufb, sema, semb,
             semt):
    cid = lax.axis_index("c")
    sid = lax.axis_index("s")
    wid = sid * 2 + cid
    b = wid // 2
    h = wid % 2
    st0 = h * WROWS

    sg = pltpu.async_copy(gidx.at[wid], gidxv, semt)
    sc = pltpu.async_copy(cidx.at[wid], cidxv, semt)
    sg.wait()
    sc.wait()

    ga = pltpu.async_copy(he.at[gidxv.at[0]], bufa, sema)
    gb = pltpu.async_copy(hc.at[cidxv.at[0]], bufb, semb)
    ga.wait()
    wa = pltpu.async_copy(bufa, out.at[b, pl.ds(st0, CH), pl.ds(0, F)], sema)
    gb.wait()
    wb = pltpu.async_copy(bufb, out.at[b, pl.ds(st0, CH), pl.ds(F, F)], semb)
    wa.wait()
    ga = pltpu.async_copy(he.at[gidxv.at[1]], bufa, sema)
    wb.wait()
    gb = pltpu.async_copy(hc.at[cidxv.at[1]], bufb, semb)
    ga.wait()
    wa = pltpu.async_copy(bufa.at[pl.ds(0, C1)],
                          out.at[b, pl.ds(st0 + CH, C1), pl.ds(0, F)], sema)
    gb.wait()
    wb = pltpu.async_copy(bufb.at[pl.ds(0, C1)],
                          out.at[b, pl.ds(st0 + CH, C1), pl.ds(F, F)], semb)
    wa.wait()
    wb.wait()


def _tc_tail_body(pe_ref, pos_ref, etab_ref, sele_ref, selrc_ref, big_ref,
                  out_ref):
    pe = pe_ref[0]
    am = jnp.argmax(pe, axis=-1)
    onehot = (lax.broadcasted_iota(jnp.int32, (S, TAGS), 1)
              == am[:, None]).astype(jnp.float32)
    emo64 = lax.dot_general(onehot, etab_ref[...], (((1,), (0,)), ((), ())),
                            precision=lax.Precision.HIGHEST)
    emo_pairs = lax.dot_general(sele_ref[...], emo64,
                                (((1,), (0,)), ((), ())),
                                precision=lax.Precision.HIGHEST)
    rel_pairs = lax.dot_general(selrc_ref[...], pos_ref[...],
                                (((1,), (0,)), ((), ())),
                                precision=lax.Precision.HIGHEST)
    pad = jnp.zeros((NPAIR, 128 - EDIM - PDIM), jnp.float32)
    out_ref[...] = jnp.concatenate([emo_pairs, rel_pairs, pad], axis=1)[None]


def _tc_edge_body(he_ref, hc_ref, big_ref, out_ref):
    out_ref[0, :, 0:F] = jnp.broadcast_to(he_ref[0, S - 1][None], (8, F))
    hc8 = jnp.concatenate([hc_ref[0, S - 4:S], hc_ref[0, S - 4:S]], axis=0)
    out_ref[0, :, F:2 * F] = hc8


def kernel(doc_sents_he, doc_sents_hc, pred_emo, pos_emb_weight,
           emo_emb_weight):
    he2 = doc_sents_he.reshape(B * S, F)
    hc2 = doc_sents_hc.reshape(B * S, F)

    mesh = plsc.VectorSubcoreMesh(core_axis_name="c", subcore_axis_name="s")
    scfn = pl.kernel(
        _sc_body,
        out_type=jax.ShapeDtypeStruct((B, NPAIR, OUTW), jnp.float32),
        mesh=mesh,
        scratch_types=[
            pltpu.VMEM((2, CH), jnp.int32),
            pltpu.VMEM((2, CH), jnp.int32),
            pltpu.VMEM((CH, F), jnp.float32),
            pltpu.VMEM((CH, F), jnp.float32),
            pltpu.SemaphoreType.DMA,
            pltpu.SemaphoreType.DMA,
            pltpu.SemaphoreType.DMA,
        ],
        compiler_params=pltpu.CompilerParams(needs_layout_passes=False),
    )
    big = scfn(he2, hc2, jnp.asarray(GIDX), jnp.asarray(CIDX))

    big = pl.pallas_call(
        _tc_tail_body,
        grid=(B,),
        in_specs=[
            pl.BlockSpec((1, S, TAGS), lambda b: (b, 0, 0)),
            pl.BlockSpec((TAGS, PDIM), lambda b: (0, 0)),
            pl.BlockSpec((TAGS, EDIM), lambda b: (0, 0)),
            pl.BlockSpec((NPAIR, S), lambda b: (0, 0)),
            pl.BlockSpec((NPAIR, TAGS), lambda b: (0, 0)),
            pl.BlockSpec(memory_space=pl.ANY),
        ],
        out_specs=pl.BlockSpec((1, NPAIR, 128), lambda b: (b, 0, TAIL0 // 128)),
        out_shape=jax.ShapeDtypeStruct((B, NPAIR, OUTW), jnp.float32),
        input_output_aliases={5: 0},
    )(pred_emo, pos_emb_weight, emo_emb_weight, jnp.asarray(SEL_E),
      jnp.asarray(SELR_COEFF), big)

    big = pl.pallas_call(
        _tc_edge_body,
        grid=(B,),
        in_specs=[
            pl.BlockSpec((1, S, F), lambda b: (b, 0, 0)),
            pl.BlockSpec((1, S, F), lambda b: (b, 0, 0)),
            pl.BlockSpec(memory_space=pl.ANY),
        ],
        out_specs=pl.BlockSpec((1, 8, 2 * F),
                               lambda b: (b, SC_ROWS // 8, 0)),
        out_shape=jax.ShapeDtypeStruct((B, NPAIR, OUTW), jnp.float32),
        input_output_aliases={2: 0},
    )(doc_sents_he, doc_sents_hc, big)

    return (big, jnp.asarray(EMO_CAU))


# speedup vs baseline: 5.8933x; 2.0483x over previous
import numpy as np
import jax
import jax.numpy as jnp
from jax import lax
from jax.experimental import pallas as pl
from jax.experimental.pallas import tpu as pltpu
from jax.experimental.pallas import tpu_sc as plsc

B = 16
S = 64
K = 3
F = 384
EDIM = 32
PDIM = 32
TAGS = 7
OUTW = 2 * F + EDIM + PDIM
TAIL0 = 2 * F

_base = np.arange(1, S + 1)
_emo = np.repeat(_base, S)
_cau = np.tile(_base, S)
_rel = _cau - _emo
_msk = np.abs(_rel) <= K
I_P = (_emo[_msk] - 1).astype(np.int32)
J_P = (_cau[_msk] - 1).astype(np.int32)
R_P = (_rel[_msk] + K).astype(np.int32)
NPAIR = int(I_P.shape[0])
EMO_CAU = np.stack([_emo[_msk], _cau[_msk]], axis=1).astype(np.int32)

_u = np.arange(2 * K + 1)
_counts = (S - np.abs(_u - K)).astype(np.float64)
COEFF = (np.exp(-((_u[:, None] - _u[None, :]) ** 2).astype(np.float64))
         * _counts[None, :])

SEL_E = np.zeros((NPAIR, S), np.float32)
SEL_E[np.arange(NPAIR), I_P] = 1.0
_selr = np.zeros((NPAIR, TAGS), np.float64)
_selr[np.arange(NPAIR), R_P] = 1.0
SELR_COEFF = (_selr @ COEFF).astype(np.float32)

NW = 32
PPW = 14
PB = 7
W_START = np.minimum(np.arange(NW) * PPW, NPAIR - PPW)

GIDX = np.zeros((NW, PPW, B), np.int32)
CIDX = np.zeros((NW, PPW, B), np.int32)
_bs = np.arange(B, dtype=np.int32) * S
for _w in range(NW):
    for _k in range(PPW):
        _p = W_START[_w] + _k
        GIDX[_w, _k] = _bs + I_P[_p]
        CIDX[_w, _k] = _bs + J_P[_p]


def _sc_body(he, hc, gidx, cidx, out, gidxv, cidxv, bufa, bufb, sema, semb,
             semt):
    cid = lax.axis_index("c")
    sid = lax.axis_index("s")
    wid = sid * 2 + cid
    s0 = jnp.minimum(wid * PPW, NPAIR - PPW)

    sg = pltpu.async_copy(gidx.at[wid], gidxv, semt)
    sc = pltpu.async_copy(cidx.at[wid], cidxv, semt)
    sg.wait()
    sc.wait()

    def fire(idxv, blk, buf, sem):
        return [pltpu.async_copy(he.at[idxv.at[blk * PB + k]], buf.at[k], sem)
                for k in range(PB)]

    def fire_c(idxv, blk, buf, sem):
        return [pltpu.async_copy(hc.at[idxv.at[blk * PB + k]], buf.at[k], sem)
                for k in range(PB)]

    ga = fire(gidxv, 0, bufa, sema)
    gb = fire_c(cidxv, 0, bufb, semb)
    for d in ga:
        d.wait()
    wa = pltpu.async_copy(bufa, out.at[pl.ds(s0, PB), :, pl.ds(0, F)], sema)
    for d in gb:
        d.wait()
    wb = pltpu.async_copy(bufb, out.at[pl.ds(s0, PB), :, pl.ds(F, F)], semb)
    wa.wait()
    ga = fire(gidxv, 1, bufa, sema)
    wb.wait()
    gb = fire_c(cidxv, 1, bufb, semb)
    for d in ga:
        d.wait()
    wa = pltpu.async_copy(bufa, out.at[pl.ds(s0 + PB, PB), :, pl.ds(0, F)],
                          sema)
    for d in gb:
        d.wait()
    wb = pltpu.async_copy(bufb, out.at[pl.ds(s0 + PB, PB), :, pl.ds(F, F)],
                          semb)
    wa.wait()
    wb.wait()


def _tc_tail_body(pe_ref, pos_ref, etab_ref, sele_ref, selrc_ref, big_ref,
                  out_ref):
    pe = pe_ref[...]
    am = jnp.argmax(pe, axis=-1)
    onehot = (lax.broadcasted_iota(jnp.int32, (B, S, TAGS), 2)
              == am[:, :, None]).astype(jnp.float32)
    emo_all = lax.dot_general(onehot, etab_ref[...], (((2,), (0,)), ((), ())),
                              precision=lax.Precision.HIGHEST)
    emo_pairs = lax.dot_general(sele_ref[...], emo_all,
                                (((1,), (1,)), ((), ())),
                                precision=lax.Precision.HIGHEST)
    rel = lax.dot_general(selrc_ref[...], pos_ref[...],
                          (((1,), (0,)), ((), ())),
                          precision=lax.Precision.HIGHEST)
    rel_pairs = jnp.broadcast_to(rel[:, None, :], (NPAIR, B, PDIM))
    pad = jnp.zeros((NPAIR, B, 128 - EDIM - PDIM), jnp.float32)
    out_ref[...] = jnp.concatenate([emo_pairs, rel_pairs, pad], axis=2)


def kernel(doc_sents_he, doc_sents_hc, pred_emo, pos_emb_weight,
           emo_emb_weight):
    he2 = doc_sents_he.reshape(B * S, F)
    hc2 = doc_sents_hc.reshape(B * S, F)

    mesh = plsc.VectorSubcoreMesh(core_axis_name="c", subcore_axis_name="s")
    scfn = pl.kernel(
        _sc_body,
        out_type=jax.ShapeDtypeStruct((NPAIR, B, OUTW), jnp.float32),
        mesh=mesh,
        scratch_types=[
            pltpu.VMEM((PPW, B), jnp.int32),
            pltpu.VMEM((PPW, B), jnp.int32),
            pltpu.VMEM((PB, B, F), jnp.float32),
            pltpu.VMEM((PB, B, F), jnp.float32),
            pltpu.SemaphoreType.DMA,
            pltpu.SemaphoreType.DMA,
            pltpu.SemaphoreType.DMA,
        ],
        compiler_params=pltpu.CompilerParams(needs_layout_passes=False),
    )
    big = scfn(he2, hc2, jnp.asarray(GIDX), jnp.asarray(CIDX))

    big = pl.pallas_call(
        _tc_tail_body,
        grid=(1,),
        in_specs=[
            pl.BlockSpec((B, S, TAGS), lambda i: (0, 0, 0)),
            pl.BlockSpec((TAGS, PDIM), lambda i: (0, 0)),
            pl.BlockSpec((TAGS, EDIM), lambda i: (0, 0)),
            pl.BlockSpec((NPAIR, S), lambda i: (0, 0)),
            pl.BlockSpec((NPAIR, TAGS), lambda i: (0, 0)),
            pl.BlockSpec(memory_space=pl.ANY),
        ],
        out_specs=pl.BlockSpec((NPAIR, B, 128), lambda i: (0, 0, TAIL0 // 128)),
        out_shape=jax.ShapeDtypeStruct((NPAIR, B, OUTW), jnp.float32),
        input_output_aliases={5: 0},
    )(pred_emo, pos_emb_weight, emo_emb_weight, jnp.asarray(SEL_E),
      jnp.asarray(SELR_COEFF), big)

    couples = jnp.transpose(big, (1, 0, 2))
    return (couples, jnp.asarray(EMO_CAU))


# speedup vs baseline: 6.4999x; 1.1029x over previous
import numpy as np
import jax
import jax.numpy as jnp
from jax import lax
from jax.experimental import pallas as pl
from jax.experimental.pallas import tpu as pltpu
from jax.experimental.pallas import tpu_sc as plsc

B = 16
S = 64
K = 3
F = 384
EDIM = 32
PDIM = 32
TAGS = 7
OUTW = 2 * F + EDIM + PDIM
TAIL0 = 2 * F

_base = np.arange(1, S + 1)
_emo = np.repeat(_base, S)
_cau = np.tile(_base, S)
_rel = _cau - _emo
_msk = np.abs(_rel) <= K
I_P = (_emo[_msk] - 1).astype(np.int32)
J_P = (_cau[_msk] - 1).astype(np.int32)
R_P = (_rel[_msk] + K).astype(np.int32)
NPAIR = int(I_P.shape[0])
EMO_CAU = np.stack([_emo[_msk], _cau[_msk]], axis=1).astype(np.int32)

_u = np.arange(2 * K + 1)
_counts = (S - np.abs(_u - K)).astype(np.float64)
COEFF = (np.exp(-((_u[:, None] - _u[None, :]) ** 2).astype(np.float64))
         * _counts[None, :])

SEL_E = np.zeros((NPAIR, S), np.float32)
SEL_E[np.arange(NPAIR), I_P] = 1.0
_selr = np.zeros((NPAIR, TAGS), np.float64)
_selr[np.arange(NPAIR), R_P] = 1.0
SELR_COEFF = (_selr @ COEFF).astype(np.float32)

NW = 32
PPW = 14
PB = 7
USLOT = 4
W_START = np.minimum(np.arange(NW) * PPW, NPAIR - PPW)

GIDXU = np.zeros((NW, USLOT * B), np.int32)
CIDX = np.zeros((NW, PPW, B), np.int32)
_bs = np.arange(B, dtype=np.int32) * S
for _w in range(NW):
    _i0 = I_P[W_START[_w]]
    for _m in range(USLOT):
        GIDXU[_w, _m * B:(_m + 1) * B] = _bs + min(_i0 + _m, S - 1)
    for _k in range(PPW):
        CIDX[_w, _k] = _bs + J_P[W_START[_w] + _k]

def _isent(p):
    lo = jnp.where(p < 4, 0, jnp.where(p < 9, 1, 2))
    hi = jnp.where(p < 427, 61, jnp.where(p < 432, 62, 63))
    return jnp.where(p < 15, lo, jnp.where(p < 421, (p - 15) // 7 + 3, hi))


def _sc_body(he, hc, gidxu, cidx, out, gidxuv, cidxv, ubuf, bufb, bufb2,
             semu, semb, semb2, semt):
    cid = lax.axis_index("c")
    sid = lax.axis_index("s")
    wid = sid * 2 + cid
    s0 = jnp.minimum(wid * PPW, NPAIR - PPW)
    i0 = _isent(s0)

    sg = pltpu.async_copy(gidxu.at[wid], gidxuv, semt)
    sc = pltpu.async_copy(cidx.at[wid], cidxv, semt)
    sg.wait()
    sc.wait()

    gu = pltpu.async_copy(he.at[gidxuv], ubuf, semu)
    gb0 = [pltpu.async_copy(hc.at[cidxv.at[k]], bufb.at[k], semb)
           for k in range(PB)]
    gb1 = [pltpu.async_copy(hc.at[cidxv.at[PB + k]], bufb2.at[k], semb2)
           for k in range(PB)]

    gu.wait()
    hw = []
    for k in range(PPW):
        m = _isent(s0 + k) - i0
        hw.append(pltpu.async_copy(
            ubuf.at[pl.ds(m * B, B)],
            out.at[s0 + k, :, pl.ds(0, F)], semu))
    for d in gb0:
        d.wait()
    wb0 = pltpu.async_copy(bufb, out.at[pl.ds(s0, PB), :, pl.ds(F, F)], semb)
    for d in gb1:
        d.wait()
    wb1 = pltpu.async_copy(bufb2, out.at[pl.ds(s0 + PB, PB), :, pl.ds(F, F)],
                           semb2)
    for d in hw:
        d.wait()
    wb0.wait()
    wb1.wait()


def _tc_tail_body(pe_ref, pos_ref, etab_ref, sele_ref, selrc_ref, big_ref,
                  out_ref):
    pe = pe_ref[...]
    am = jnp.argmax(pe, axis=-1)
    onehot = (lax.broadcasted_iota(jnp.int32, (B, S, TAGS), 2)
              == am[:, :, None]).astype(jnp.float32)
    emo_all = lax.dot_general(onehot, etab_ref[...], (((2,), (0,)), ((), ())),
                              precision=lax.Precision.HIGHEST)
    emo_pairs = lax.dot_general(sele_ref[...], emo_all,
                                (((1,), (1,)), ((), ())),
                                precision=lax.Precision.HIGHEST)
    rel = lax.dot_general(selrc_ref[...], pos_ref[...],
                          (((1,), (0,)), ((), ())),
                          precision=lax.Precision.HIGHEST)
    rel_pairs = jnp.broadcast_to(rel[:, None, :], (NPAIR, B, PDIM))
    pad = jnp.zeros((NPAIR, B, 128 - EDIM - PDIM), jnp.float32)
    out_ref[...] = jnp.concatenate([emo_pairs, rel_pairs, pad], axis=2)


def kernel(doc_sents_he, doc_sents_hc, pred_emo, pos_emb_weight,
           emo_emb_weight):
    he2 = doc_sents_he.reshape(B * S, F)
    hc2 = doc_sents_hc.reshape(B * S, F)

    mesh = plsc.VectorSubcoreMesh(core_axis_name="c", subcore_axis_name="s")
    scfn = pl.kernel(
        _sc_body,
        out_type=jax.ShapeDtypeStruct((NPAIR, B, OUTW), jnp.float32),
        mesh=mesh,
        scratch_types=[
            pltpu.VMEM((USLOT * B,), jnp.int32),
            pltpu.VMEM((PPW, B), jnp.int32),
            pltpu.VMEM((USLOT * B, F), jnp.float32),
            pltpu.VMEM((PB, B, F), jnp.float32),
            pltpu.VMEM((PB, B, F), jnp.float32),
            pltpu.SemaphoreType.DMA,
            pltpu.SemaphoreType.DMA,
            pltpu.SemaphoreType.DMA,
            pltpu.SemaphoreType.DMA,
        ],
        compiler_params=pltpu.CompilerParams(needs_layout_passes=False),
    )
    big = scfn(he2, hc2, jnp.asarray(GIDXU), jnp.asarray(CIDX))

    big = pl.pallas_call(
        _tc_tail_body,
        grid=(1,),
        in_specs=[
            pl.BlockSpec((B, S, TAGS), lambda i: (0, 0, 0)),
            pl.BlockSpec((TAGS, PDIM), lambda i: (0, 0)),
            pl.BlockSpec((TAGS, EDIM), lambda i: (0, 0)),
            pl.BlockSpec((NPAIR, S), lambda i: (0, 0)),
            pl.BlockSpec((NPAIR, TAGS), lambda i: (0, 0)),
            pl.BlockSpec(memory_space=pl.ANY),
        ],
        out_specs=pl.BlockSpec((NPAIR, B, 128), lambda i: (0, 0, TAIL0 // 128)),
        out_shape=jax.ShapeDtypeStruct((NPAIR, B, OUTW), jnp.float32),
        input_output_aliases={5: 0},
    )(pred_emo, pos_emb_weight, emo_emb_weight, jnp.asarray(SEL_E),
      jnp.asarray(SELR_COEFF), big)

    couples = jnp.transpose(big, (1, 0, 2))
    return (couples, jnp.asarray(EMO_CAU))


# speedup vs baseline: 6.7521x; 1.0388x over previous
import numpy as np
import jax
import jax.numpy as jnp
from jax import lax
from jax.experimental import pallas as pl
from jax.experimental.pallas import tpu as pltpu
from jax.experimental.pallas import tpu_sc as plsc

B = 16
S = 64
K = 3
F = 384
EDIM = 32
PDIM = 32
TAGS = 7
OUTW = 2 * F + EDIM + PDIM
TAIL0 = 2 * F

_base = np.arange(1, S + 1)
_emo = np.repeat(_base, S)
_cau = np.tile(_base, S)
_rel = _cau - _emo
_msk = np.abs(_rel) <= K
I_P = (_emo[_msk] - 1).astype(np.int32)
J_P = (_cau[_msk] - 1).astype(np.int32)
R_P = (_rel[_msk] + K).astype(np.int32)
NPAIR = int(I_P.shape[0])
EMO_CAU = np.stack([_emo[_msk], _cau[_msk]], axis=1).astype(np.int32)

_u = np.arange(2 * K + 1)
_counts = (S - np.abs(_u - K)).astype(np.float64)
COEFF = (np.exp(-((_u[:, None] - _u[None, :]) ** 2).astype(np.float64))
         * _counts[None, :])

SEL_E = np.zeros((NPAIR, S), np.float32)
SEL_E[np.arange(NPAIR), I_P] = 1.0
_selr = np.zeros((NPAIR, TAGS), np.float64)
_selr[np.arange(NPAIR), R_P] = 1.0
SELR_COEFF = (_selr @ COEFF).astype(np.float32)

NW = 32
PPW = 14
USLOT = 4
CSLOT = 10


def _isent(p):
    lo = jnp.where(p < 4, 0, jnp.where(p < 9, 1, 2))
    hi = jnp.where(p < 427, 61, jnp.where(p < 432, 62, 63))
    return jnp.where(p < 15, lo, jnp.where(p < 421, (p - 15) // 7 + 3, hi))


def _gstart(i):
    lo = jnp.where(i == 0, 0, jnp.where(i == 1, 4, 9))
    hi = jnp.where(i == 62, 427, 432)
    return jnp.where(i < 3, lo, jnp.where(i < 62, 15 + 7 * (i - 3), hi))


def _jsent(p, i):
    return jnp.maximum(0, i - K) + (p - _gstart(i))


def _sc_body(he, hc, out, gidxuv, cidxv, ubuf, cbuf, semu, semc):
    cid = lax.axis_index("c")
    sid = lax.axis_index("s")
    wid = sid * 2 + cid
    s0 = jnp.minimum(wid * PPW, NPAIR - PPW)
    i0 = _isent(s0)
    jmin = jnp.maximum(0, i0 - K)

    bvec = jnp.arange(16, dtype=jnp.int32) * S
    for m in range(USLOT):
        gidxuv[pl.ds(m * B, B)] = bvec + jnp.minimum(i0 + m, S - 1)
    for m in range(CSLOT):
        cidxv[m // 5, pl.ds((m % 5) * B, B)] = bvec + jnp.minimum(
            jmin + m, S - 1)

    gu = pltpu.async_copy(he.at[gidxuv], ubuf, semu)
    gc0 = pltpu.async_copy(hc.at[cidxv.at[0]], cbuf.at[pl.ds(0, 5 * B)], semc)
    gc1 = pltpu.async_copy(hc.at[cidxv.at[1]], cbuf.at[pl.ds(5 * B, 5 * B)],
                           semc)

    ms, mcs = [], []
    for k in range(PPW):
        i = _isent(s0 + k)
        ms.append(i - i0)
        mcs.append(_jsent(s0 + k, i) - jmin)

    gu.wait()
    hw = []
    for k in range(PPW):
        hw.append(pltpu.async_copy(
            ubuf.at[pl.ds(ms[k] * B, B)],
            out.at[s0 + k, :, pl.ds(0, F)], semu))
    gc0.wait()
    gc1.wait()
    for k in range(PPW):
        hw.append(pltpu.async_copy(
            cbuf.at[pl.ds(mcs[k] * B, B)],
            out.at[s0 + k, :, pl.ds(F, F)], semc))
    for d in hw:
        d.wait()


def _tc_tail_body(pe_ref, pos_ref, etab_ref, sele_ref, selrc_ref, big_ref,
                  out_ref):
    pe = pe_ref[...]
    am = jnp.argmax(pe, axis=-1)
    onehot = (lax.broadcasted_iota(jnp.int32, (B, S, TAGS), 2)
              == am[:, :, None]).astype(jnp.float32)
    emo_all = lax.dot_general(onehot, etab_ref[...], (((2,), (0,)), ((), ())),
                              precision=lax.Precision.HIGHEST)
    emo_pairs = lax.dot_general(sele_ref[...], emo_all,
                                (((1,), (1,)), ((), ())),
                                precision=lax.Precision.HIGHEST)
    rel = lax.dot_general(selrc_ref[...], pos_ref[...],
                          (((1,), (0,)), ((), ())),
                          precision=lax.Precision.HIGHEST)
    rel_pairs = jnp.broadcast_to(rel[:, None, :], (NPAIR, B, PDIM))
    pad = jnp.zeros((NPAIR, B, 128 - EDIM - PDIM), jnp.float32)
    out_ref[...] = jnp.concatenate([emo_pairs, rel_pairs, pad], axis=2)


def kernel(doc_sents_he, doc_sents_hc, pred_emo, pos_emb_weight,
           emo_emb_weight):
    he2 = doc_sents_he.reshape(B * S, F)
    hc2 = doc_sents_hc.reshape(B * S, F)

    mesh = plsc.VectorSubcoreMesh(core_axis_name="c", subcore_axis_name="s")
    scfn = pl.kernel(
        _sc_body,
        out_type=jax.ShapeDtypeStruct((NPAIR, B, OUTW), jnp.float32),
        mesh=mesh,
        scratch_types=[
            pltpu.VMEM((USLOT * B,), jnp.int32),
            pltpu.VMEM((2, 5 * B), jnp.int32),
            pltpu.VMEM((USLOT * B, F), jnp.float32),
            pltpu.VMEM((CSLOT * B, F), jnp.float32),
            pltpu.SemaphoreType.DMA,
            pltpu.SemaphoreType.DMA,
        ],
        compiler_params=pltpu.CompilerParams(needs_layout_passes=False),
    )
    big = scfn(he2, hc2)

    big = pl.pallas_call(
        _tc_tail_body,
        grid=(1,),
        in_specs=[
            pl.BlockSpec((B, S, TAGS), lambda i: (0, 0, 0)),
            pl.BlockSpec((TAGS, PDIM), lambda i: (0, 0)),
            pl.BlockSpec((TAGS, EDIM), lambda i: (0, 0)),
            pl.BlockSpec((NPAIR, S), lambda i: (0, 0)),
            pl.BlockSpec((NPAIR, TAGS), lambda i: (0, 0)),
            pl.BlockSpec(memory_space=pl.ANY),
        ],
        out_specs=pl.BlockSpec((NPAIR, B, 128), lambda i: (0, 0, TAIL0 // 128)),
        out_shape=jax.ShapeDtypeStruct((NPAIR, B, OUTW), jnp.float32),
        input_output_aliases={5: 0},
    )(pred_emo, pos_emb_weight, emo_emb_weight, jnp.asarray(SEL_E),
      jnp.asarray(SELR_COEFF), big)

    couples = jnp.transpose(big, (1, 0, 2))
    return (couples, jnp.asarray(EMO_CAU))
